# baseline (speedup 1.0000x reference)
import functools

import jax
import jax.numpy as jnp
from jax import lax
from jax.experimental import pallas as pl
from jax.experimental.pallas import tpu as pltpu
from jax.experimental.pallas import tpu_sc as plsc

BATCH = 16384
EMBED_DIM = 32
PACK = 4
WIDE = PACK * EMBED_DIM
NUM_CORES = 2
NUM_SUBCORES = 16
NUM_WORKERS = NUM_CORES * NUM_SUBCORES
B_PER_W = BATCH // NUM_WORKERS
CHUNK = 256
NUM_ROUNDS = B_PER_W // CHUNK
LANES = 16


def kernel(indices, table):
    idx = indices.astype(jnp.int32)
    tabw = table.reshape(table.shape[0] // PACK, WIDE)
    mesh = plsc.VectorSubcoreMesh(core_axis_name="c", subcore_axis_name="s")

    @functools.partial(
        pl.kernel,
        mesh=mesh,
        compiler_params=pltpu.CompilerParams(needs_layout_passes=False),
        out_type=jax.ShapeDtypeStruct((BATCH, EMBED_DIM), jnp.float32),
        scratch_types=[
            pltpu.VMEM((B_PER_W,), jnp.int32),
            pltpu.VMEM((B_PER_W,), jnp.int32),
            pltpu.VMEM((CHUNK, WIDE), jnp.float32),
            pltpu.VMEM((CHUNK, EMBED_DIM), jnp.float32),
            pltpu.SemaphoreType.DMA,
        ],
    )
    def gather_kernel(tab_hbm, idx_hbm, out_hbm, idx_v, q_v, rows_v, out_v, sem):
        wid = lax.axis_index("s") * NUM_CORES + lax.axis_index("c")
        base = wid * B_PER_W
        pltpu.sync_copy(idx_hbm.at[pl.ds(base, B_PER_W)], idx_v)

        @pl.loop(0, B_PER_W, step=LANES)
        def _shift(i):
            v = idx_v[pl.ds(i, LANES)]
            q_v[pl.ds(i, LANES)] = lax.shift_right_logical(v, 2)

        @pl.loop(0, NUM_ROUNDS)
        def _round(rd):
            o = rd * CHUNK
            pltpu.async_copy(
                tab_hbm.at[q_v.at[pl.ds(o, CHUNK)]], rows_v, sem
            ).wait()

            @pl.loop(0, CHUNK, step=LANES)
            def _group(j):
                rem = lax.bitwise_and(
                    idx_v[pl.ds(o + j, LANES)], jnp.int32(3)
                )
                inner_base = rem * EMBED_DIM
                jvec = lax.iota(jnp.int32, LANES) + j

                @pl.loop(0, EMBED_DIM)
                def _col(c):
                    cvec = lax.iota(jnp.int32, LANES) * 0 + c
                    vals = plsc.load_gather(rows_v, [jvec, inner_base + cvec])
                    plsc.store_scatter(out_v, [jvec, cvec], vals)

            pltpu.sync_copy(out_v, out_hbm.at[pl.ds(base + o, CHUNK)])

    return gather_kernel(tabw, idx)
